# baseline (speedup 1.0000x reference)
import functools

import jax
import jax.numpy as jnp
from jax.experimental import pallas as pl
from jax.experimental.pallas import tpu as pltpu

_ALPHA = 0.7
_BETA = 0.3

_Q_TILE = 1568
_N_TILE = 1000


def _body(q_ref, neg_ref, pos_ref, o_ref, mn_ref, mp_ref, *, nn):
    j = pl.program_id(1)
    q = q_ref[...]
    qf = q.astype(jnp.float32)
    qn = jnp.sum(qf * qf, axis=1, keepdims=True)

    def tile_min(b_ref):
        b = b_ref[...].astype(jnp.float8_e4m3fn)
        dot = jax.lax.dot_general(
            q, b, (((1,), (1,)), ((), ())), preferred_element_type=jnp.float32
        )
        bf = b.astype(jnp.float32)
        bn = jnp.sum(bf * bf, axis=1)
        d2 = jnp.maximum(qn + bn[None, :] - 2.0 * dot, 0.0)
        return jnp.min(d2, axis=1, keepdims=True)

    tn = tile_min(neg_ref)
    tp = tile_min(pos_ref)

    @pl.when(j == 0)
    def _init():
        mn_ref[...] = tn
        mp_ref[...] = tp

    @pl.when(j > 0)
    def _acc():
        mn_ref[...] = jnp.minimum(mn_ref[...], tn)
        mp_ref[...] = jnp.minimum(mp_ref[...], tp)

    @pl.when(j == nn - 1)
    def _emit():
        o_ref[...] = _ALPHA * jnp.sqrt(mn_ref[...] + 1e-12) - _BETA * jnp.sqrt(
            mp_ref[...] + 1e-12
        )


def kernel(queries, neg_bank, pos_bank):
    nq_rows, d = queries.shape
    n = neg_bank.shape[0]
    nq = nq_rows // _Q_TILE
    nn = n // _N_TILE
    q8 = queries.astype(jnp.float8_e4m3fn)
    out = pl.pallas_call(
        functools.partial(_body, nn=nn),
        grid=(nq, nn),
        in_specs=[
            pl.BlockSpec((_Q_TILE, d), lambda i, j: (i, 0)),
            pl.BlockSpec((_N_TILE, d), lambda i, j: (j, 0)),
            pl.BlockSpec((_N_TILE, d), lambda i, j: (j, 0)),
        ],
        out_specs=pl.BlockSpec((_Q_TILE, 1), lambda i, j: (i, 0)),
        out_shape=jax.ShapeDtypeStruct((nq_rows, 1), jnp.float32),
        scratch_shapes=[
            pltpu.VMEM((_Q_TILE, 1), jnp.float32),
            pltpu.VMEM((_Q_TILE, 1), jnp.float32),
        ],
        compiler_params=pltpu.CompilerParams(
            dimension_semantics=("parallel", "arbitrary"),
        ),
    )(q8, neg_bank, pos_bank)
    return out[:, 0]
